# speedup vs baseline: 1.0087x; 1.0087x over previous
import functools

import jax
import jax.numpy as jnp
from jax.experimental import pallas as pl

_NUM_EXPERTS = 64
_K = 8
_NEG_INF = float("-inf")


def _gate_body(x_ref, w_ref, b_ref, sm_ref, idx_ref):
    logits = jnp.dot(x_ref[...], w_ref[...], preferred_element_type=jnp.float32)
    logits = logits + b_ref[...]

    lane = jax.lax.broadcasted_iota(jnp.int32, logits.shape, 1)
    l = logits
    vals = []
    idxs = []
    for _ in range(_K):
        m = jnp.max(l, axis=1, keepdims=True)
        am = jnp.min(jnp.where(l == m, lane, _NUM_EXPERTS), axis=1, keepdims=True)
        vals.append(m)
        idxs.append(am)
        l = jnp.where(lane == am, _NEG_INF, l)

    v = jnp.concatenate(vals, axis=1)
    ix = jnp.concatenate(idxs, axis=1)
    e = jnp.exp(v - v[:, 0:1])
    sm_ref[...] = e / jnp.sum(e, axis=1, keepdims=True)
    idx_ref[...] = ix


@jax.jit
def kernel(x, W, b):
    n_tokens, d = x.shape
    block = 512
    grid = (n_tokens // block,)
    wt = W.T
    b2 = b.reshape(1, _NUM_EXPERTS)
    out_shapes = (
        jax.ShapeDtypeStruct((n_tokens, _K), jnp.float32),
        jax.ShapeDtypeStruct((n_tokens, _K), jnp.int32),
    )
    sm, idx = pl.pallas_call(
        _gate_body,
        grid=grid,
        in_specs=[
            pl.BlockSpec((block, d), lambda i: (i, 0)),
            pl.BlockSpec((d, _NUM_EXPERTS), lambda i: (0, 0)),
            pl.BlockSpec((1, _NUM_EXPERTS), lambda i: (0, 0)),
        ],
        out_specs=(
            pl.BlockSpec((block, _K), lambda i: (i, 0)),
            pl.BlockSpec((block, _K), lambda i: (i, 0)),
        ),
        out_shape=out_shapes,
    )(x, wt, b2)
    return sm, idx


# speedup vs baseline: 1.4557x; 1.4431x over previous
import jax
import jax.numpy as jnp
from jax.experimental import pallas as pl

_NUM_EXPERTS = 64
_K = 8
_NEG_INF = float("-inf")


def _gate_body(x_ref, w_ref, b_ref, sm_ref, idx_ref):
    logits = jnp.dot(x_ref[...], w_ref[...], preferred_element_type=jnp.float32)
    logits = logits + b_ref[...]

    lt = logits.T
    srow = jax.lax.broadcasted_iota(jnp.int32, lt.shape, 0)
    vals = []
    idxs = []
    for _ in range(_K):
        m = jnp.max(lt, axis=0, keepdims=True)
        am = jnp.min(jnp.where(lt == m, srow, _NUM_EXPERTS), axis=0, keepdims=True)
        vals.append(m)
        idxs.append(am)
        lt = jnp.where(srow == am, _NEG_INF, lt)

    v = jnp.concatenate(vals, axis=0)
    ix = jnp.concatenate(idxs, axis=0)
    e = jnp.exp(v - v[0:1, :])
    sm = e / jnp.sum(e, axis=0, keepdims=True)
    sm_ref[...] = sm.T
    idx_ref[...] = ix.T


@jax.jit
def kernel(x, W, b):
    n_tokens, d = x.shape
    block = 512
    grid = (n_tokens // block,)
    wt = W.T
    b2 = b.reshape(1, _NUM_EXPERTS)
    out_shapes = (
        jax.ShapeDtypeStruct((n_tokens, _K), jnp.float32),
        jax.ShapeDtypeStruct((n_tokens, _K), jnp.int32),
    )
    sm, idx = pl.pallas_call(
        _gate_body,
        grid=grid,
        in_specs=[
            pl.BlockSpec((block, d), lambda i: (i, 0)),
            pl.BlockSpec((d, _NUM_EXPERTS), lambda i: (0, 0)),
            pl.BlockSpec((1, _NUM_EXPERTS), lambda i: (0, 0)),
        ],
        out_specs=(
            pl.BlockSpec((block, _K), lambda i: (i, 0)),
            pl.BlockSpec((block, _K), lambda i: (i, 0)),
        ),
        out_shape=out_shapes,
    )(x, wt, b2)
    return sm, idx
